# baseline (speedup 1.0000x reference)
import functools

import jax
import jax.numpy as jnp
from jax import lax
from jax.experimental import pallas as pl
from jax.experimental.pallas import tpu as pltpu
from jax.experimental.pallas import tpu_sc as plsc

D = 128
NC = 2
NS = 16
NW = NC * NS
CHUNK = 80
NBUF = 4
IB = 40


@functools.partial(jax.jit, static_argnames=("n_blocks",))
def _sc_gather(idx4, weights, n_blocks):
    B = NW * n_blocks * IB * CHUNK
    total = n_blocks * IB
    mesh = plsc.VectorSubcoreMesh(core_axis_name="c", subcore_axis_name="s")

    @functools.partial(
        pl.kernel,
        out_type=jax.ShapeDtypeStruct((B, D), jnp.float32),
        mesh=mesh,
        scratch_types=[
            pltpu.VMEM((2, IB, CHUNK), jnp.int32),
            pltpu.VMEM((2, 2 * CHUNK, D), jnp.float32),
            pltpu.VMEM_SHARED((8192, D), jnp.float32),
            pltpu.SemaphoreType.DMA((NBUF,)),
            pltpu.SemaphoreType.DMA((2,)),
            pltpu.SemaphoreType.DMA((2,)),
        ],
    )
    def k(idx_hbm, table_hbm, out_hbm, idx_v, rows_v, table_sh, gsem, ssem, isem):
        wid = lax.axis_index("s") * NC + lax.axis_index("c")
        base = wid * (total * CHUNK)

        sid = lax.axis_index("s")
        rows_per_tile = 8192 // NS
        pltpu.sync_copy(
            table_hbm.at[pl.ds(sid * rows_per_tile, rows_per_tile)],
            table_sh.at[pl.ds(sid * rows_per_tile, rows_per_tile)],
        )
        pltpu.sync_copy(idx_hbm.at[wid, 0], idx_v.at[0])
        plsc.subcore_barrier()

        def load_idx(blk, p):
            return pltpu.make_async_copy(
                idx_hbm.at[wid, blk], idx_v.at[p], isem.at[p]
            )

        def gather(p, j, s):
            dst = rows_v.at[s // 2].at[pl.ds((s % 2) * CHUNK, CHUNK)]
            return pltpu.make_async_copy(
                table_sh.at[idx_v.at[p].at[j]], dst, gsem.at[s]
            )

        def store(q, ss):
            off = base + q * (2 * CHUNK)
            return pltpu.make_async_copy(
                rows_v.at[ss], out_hbm.at[pl.ds(off, 2 * CHUNK)], ssem.at[ss]
            )

        for j in range(NBUF):
            gather(0, j, j).start()

        def body(k_, _):
            b = lax.rem(k_, 2)

            @pl.when(k_ + 1 < n_blocks)
            def _():
                load_idx(k_ + 1, 1 - b).start()

            for jp in range(IB // 2):
                j = 2 * jp
                q = k_ * (IB // 2) + jp
                ss = jp % 2
                gather(b, j, 2 * ss).wait()
                gather(b, j + 1, 2 * ss + 1).wait()
                store(q, ss).start()

                if j == IB - NBUF:
                    @pl.when(k_ + 1 < n_blocks)
                    def _():
                        load_idx(k_ + 1, 1 - b).wait()

                @pl.when(q + 2 < total // 2)
                def _():
                    store(q, ss).wait()
                    if j < IB - NBUF:
                        gather(b, j + NBUF, 2 * ss).start()
                        gather(b, j + NBUF + 1, 2 * ss + 1).start()
                    else:
                        gather(1 - b, j + NBUF - IB, 2 * ss).start()
                        gather(1 - b, j + NBUF + 1 - IB, 2 * ss + 1).start()

            return 0

        lax.fori_loop(0, n_blocks, body, 0)

        for q in range(total // 2 - 2, total // 2):
            store(q, q % 2).wait()

    return k(idx4, weights)


def kernel(detail_pos, weights):
    shape = detail_pos.shape
    flat = detail_pos.reshape(-1).astype(jnp.int32)
    n_blocks = flat.shape[0] // (NW * IB * CHUNK)
    idx4 = flat.reshape(NW, n_blocks, IB, CHUNK)
    out = _sc_gather(idx4, weights.astype(jnp.float32), n_blocks)
    return out.reshape(shape + (weights.shape[-1],))


# speedup vs baseline: 1.0145x; 1.0145x over previous
import functools

import jax
import jax.numpy as jnp
from jax import lax
from jax.experimental import pallas as pl
from jax.experimental.pallas import tpu as pltpu
from jax.experimental.pallas import tpu_sc as plsc

D = 128
NC = 2
NS = 16
NW = NC * NS
CHUNK = 80
NBUF = 4
IB = 80


@functools.partial(jax.jit, static_argnames=("n_blocks",))
def _sc_gather(idx4, weights, n_blocks):
    B = NW * n_blocks * IB * CHUNK
    total = n_blocks * IB
    mesh = plsc.VectorSubcoreMesh(core_axis_name="c", subcore_axis_name="s")

    @functools.partial(
        pl.kernel,
        out_type=jax.ShapeDtypeStruct((B, D), jnp.float32),
        mesh=mesh,
        scratch_types=[
            pltpu.VMEM((2, IB, CHUNK), jnp.int32),
            pltpu.VMEM((NBUF, CHUNK, D), jnp.float32),
            pltpu.VMEM_SHARED((8192, D), jnp.float32),
            pltpu.SemaphoreType.DMA((NBUF,)),
            pltpu.SemaphoreType.DMA((NBUF,)),
            pltpu.SemaphoreType.DMA((2,)),
        ],
    )
    def k(idx_hbm, table_hbm, out_hbm, idx_v, rows_v, table_sh, gsem, ssem, isem):
        wid = lax.axis_index("s") * NC + lax.axis_index("c")
        base = wid * (total * CHUNK)

        sid = lax.axis_index("s")
        rows_per_tile = 8192 // NS
        pltpu.sync_copy(
            table_hbm.at[pl.ds(sid * rows_per_tile, rows_per_tile)],
            table_sh.at[pl.ds(sid * rows_per_tile, rows_per_tile)],
        )
        pltpu.sync_copy(idx_hbm.at[wid, 0], idx_v.at[0])
        plsc.subcore_barrier()

        def load_idx(blk, p):
            return pltpu.make_async_copy(
                idx_hbm.at[wid, blk], idx_v.at[p], isem.at[p]
            )

        def gather(p, j, s):
            return pltpu.make_async_copy(
                table_sh.at[idx_v.at[p].at[j]], rows_v.at[s], gsem.at[s]
            )

        def store(g, s):
            off = base + g * CHUNK
            return pltpu.make_async_copy(
                rows_v.at[s], out_hbm.at[pl.ds(off, CHUNK)], ssem.at[s]
            )

        for j in range(NBUF):
            gather(0, j, j).start()

        def body(k_, _):
            b = lax.rem(k_, 2)

            @pl.when(k_ + 1 < n_blocks)
            def _():
                load_idx(k_ + 1, 1 - b).start()

            for j in range(IB):
                g = k_ * IB + j
                s = j % NBUF
                gather(b, j, s).wait()
                store(g, s).start()

                if j == IB - NBUF:
                    @pl.when(k_ + 1 < n_blocks)
                    def _():
                        load_idx(k_ + 1, 1 - b).wait()

                @pl.when(g + NBUF < total)
                def _():
                    store(g, s).wait()
                    if j < IB - NBUF:
                        gather(b, j + NBUF, s).start()
                    else:
                        gather(1 - b, j + NBUF - IB, s).start()

            return 0

        lax.fori_loop(0, n_blocks, body, 0)

        for s in range(NBUF):
            store(total - NBUF + s, s).wait()

    return k(idx4, weights)


def kernel(detail_pos, weights):
    shape = detail_pos.shape
    flat = detail_pos.reshape(-1).astype(jnp.int32)
    n_blocks = flat.shape[0] // (NW * IB * CHUNK)
    idx4 = flat.reshape(NW, n_blocks, IB, CHUNK)
    out = _sc_gather(idx4, weights.astype(jnp.float32), n_blocks)
    return out.reshape(shape + (weights.shape[-1],))


# speedup vs baseline: 1.0210x; 1.0064x over previous
import functools

import jax
import jax.numpy as jnp
from jax import lax
from jax.experimental import pallas as pl
from jax.experimental.pallas import tpu as pltpu
from jax.experimental.pallas import tpu_sc as plsc

D = 128
NC = 2
NS = 16
NW = NC * NS
CHUNK = 80
NBUF = 4
IB = 40


@functools.partial(jax.jit, static_argnames=("n_blocks",))
def _sc_gather(idx4, weights, n_blocks):
    B = NW * n_blocks * IB * CHUNK
    total = n_blocks * IB
    mesh = plsc.VectorSubcoreMesh(core_axis_name="c", subcore_axis_name="s")

    @functools.partial(
        pl.kernel,
        out_type=jax.ShapeDtypeStruct((B, D), jnp.float32),
        mesh=mesh,
        scratch_types=[
            pltpu.VMEM((2, IB, CHUNK), jnp.int32),
            pltpu.VMEM((NBUF, CHUNK, D), jnp.float32),
            pltpu.VMEM_SHARED((8192, D), jnp.float32),
            pltpu.SemaphoreType.DMA((NBUF,)),
            pltpu.SemaphoreType.DMA((NBUF,)),
            pltpu.SemaphoreType.DMA((2,)),
            pltpu.SemaphoreType.DMA,
        ],
    )
    def k(idx_hbm, table_hbm, out_hbm, idx_v, rows_v, table_sh, gsem, ssem, isem, stsem):
        wid = lax.axis_index("s") * NC + lax.axis_index("c")
        base = wid * (total * CHUNK)

        sid = lax.axis_index("s")
        rows_per_tile = 8192 // NS
        staging = pltpu.make_async_copy(
            table_hbm.at[pl.ds(sid * rows_per_tile, rows_per_tile)],
            table_sh.at[pl.ds(sid * rows_per_tile, rows_per_tile)],
            stsem,
        )
        staging.start()
        pltpu.async_copy(idx_hbm.at[wid, 0], idx_v.at[0], isem.at[0])
        staging.wait()
        plsc.subcore_barrier()
        pltpu.make_async_copy(idx_hbm.at[wid, 0], idx_v.at[0], isem.at[0]).wait()

        def load_idx(blk, p):
            return pltpu.make_async_copy(
                idx_hbm.at[wid, blk], idx_v.at[p], isem.at[p]
            )

        def gather(p, j, s):
            return pltpu.make_async_copy(
                table_sh.at[idx_v.at[p].at[j]], rows_v.at[s], gsem.at[s]
            )

        def store(g, s):
            off = base + g * CHUNK
            return pltpu.make_async_copy(
                rows_v.at[s], out_hbm.at[pl.ds(off, CHUNK)], ssem.at[s]
            )

        for j in range(NBUF):
            gather(0, j, j).start()

        def body(k_, _):
            b = lax.rem(k_, 2)

            @pl.when(k_ + 1 < n_blocks)
            def _():
                load_idx(k_ + 1, 1 - b).start()

            for j in range(IB):
                g = k_ * IB + j
                s = j % NBUF
                gather(b, j, s).wait()
                store(g, s).start()

                if j == IB - NBUF:
                    @pl.when(k_ + 1 < n_blocks)
                    def _():
                        load_idx(k_ + 1, 1 - b).wait()

                @pl.when(g + NBUF < total)
                def _():
                    store(g, s).wait()
                    if j < IB - NBUF:
                        gather(b, j + NBUF, s).start()
                    else:
                        gather(1 - b, j + NBUF - IB, s).start()

            return 0

        lax.fori_loop(0, n_blocks, body, 0)

        for s in range(NBUF):
            store(total - NBUF + s, s).wait()

    return k(idx4, weights)


def kernel(detail_pos, weights):
    shape = detail_pos.shape
    flat = detail_pos.reshape(-1).astype(jnp.int32)
    n_blocks = flat.shape[0] // (NW * IB * CHUNK)
    idx4 = flat.reshape(NW, n_blocks, IB, CHUNK)
    out = _sc_gather(idx4, weights.astype(jnp.float32), n_blocks)
    return out.reshape(shape + (weights.shape[-1],))
